# Initial kernel scaffold; baseline (speedup 1.0000x reference)
import jax
import jax.numpy as jnp
from jax.experimental import pallas as pl


def kernel(inputs, targets):
    raise NotImplementedError("write your pallas kernel here")


# speedup vs baseline: 1.8095x; 1.8095x over previous
import jax
import jax.numpy as jnp
from jax import lax
from jax.experimental import pallas as pl
from jax.experimental.pallas import tpu as pltpu
from jax.experimental.pallas import tpu_sc as plsc

B = 16384
C = 1000
NC = 2
NS = 16
NW = NC * NS
BPW = B // NW
NG = BPW // 16
ROWPAD = (C + 15) // 16 * 16

_IN_BOUNDS = "promise_in_bounds"


def _rot(x, lanes, sh):
    return x.at[(lanes + sh) & 15].get(mode=_IN_BOUNDS)


def _allsum(x, lanes):
    for sh in (8, 4, 2, 1):
        x = x + _rot(x, lanes, sh)
    return x


def _prefix_sum(x, lanes):
    zero = jnp.zeros_like(x)
    for sh in (1, 2, 4, 8):
        y = _rot(x, lanes, -sh)
        x = x + jnp.where(lanes >= sh, y, zero)
    return x


def _sc_body(flat_hbm, tgt_hbm, out_hbm,
             tgt_v, idx_t, idx_0, xt_v, x0_v, zrows_v, out_v, rowbuf_v, sem):
    wid = lax.axis_index("s") * NC + lax.axis_index("c")
    base = wid * BPW
    lanes = lax.iota(jnp.int32, 16)

    pltpu.sync_copy(tgt_hbm.at[pl.ds(base, BPW)], tgt_v)

    cursor = jnp.zeros((16,), jnp.int32)
    for g in range(NG):
        t16 = tgt_v[pl.ds(g * 16, 16)]
        rows_loc = g * 16 + lanes
        row_base = (base + rows_loc) * C
        idx_t[pl.ds(g * 16, 16)] = row_base + t16
        idx_0[pl.ds(g * 16, 16)] = row_base
        zint = (t16 == 0).astype(jnp.int32)
        pos = _prefix_sum(zint, lanes)
        zmask = t16 == 0
        zidx = jnp.where(zmask, cursor + pos - 1, 0)
        plsc.store_scatter(zrows_v, [zidx], rows_loc, mask=zmask)
        cursor = cursor + _allsum(zint, lanes)
    n0 = cursor[0]

    copies = []
    for k in range(BPW // 128):
        sl = pl.ds(k * 128, 128)
        copies.append(pltpu.async_copy(flat_hbm.at[idx_t.at[sl]], xt_v.at[sl], sem))
        copies.append(pltpu.async_copy(flat_hbm.at[idx_0.at[sl]], x0_v.at[sl], sem))
    for cp in copies:
        cp.wait()

    for g in range(NG):
        sl = pl.ds(g * 16, 16)
        out_v[sl] = 1.0 + jnp.exp(x0_v[sl] - xt_v[sl])

    def zrow_body(j, carry):
        j16 = jnp.broadcast_to(j, (16,)).astype(jnp.int32)
        row_loc = plsc.load_gather(zrows_v, [j16])[0]
        off = (base + row_loc) * C
        pltpu.sync_copy(flat_hbm.at[pl.ds(off, C)], rowbuf_v.at[pl.ds(0, C)])
        x0s = plsc.load_gather(rowbuf_v, [jnp.zeros((16,), jnp.int32)])
        acc = jnp.zeros((16,), jnp.float32)
        for k in range(ROWPAD // 16):
            v = rowbuf_v[pl.ds(k * 16, 16)]
            if (k + 1) * 16 > C:
                v = jnp.where(lanes < C - k * 16, v, -1e30)
            acc = acc + jnp.exp(v - x0s)
        tot = _allsum(acc, lanes)
        plsc.store_scatter(out_v, [jnp.broadcast_to(row_loc, (16,))], tot,
                           mask=lanes == 0)
        return carry

    lax.fori_loop(0, n0, zrow_body, 0)

    pltpu.sync_copy(out_v, out_hbm.at[pl.ds(base, BPW)])


@jax.jit
def _sc_ratios(flat, tgt):
    mesh = plsc.VectorSubcoreMesh(core_axis_name="c", subcore_axis_name="s",
                                  num_cores=NC, num_subcores=NS)
    return pl.kernel(
        _sc_body,
        out_type=jax.ShapeDtypeStruct((B,), jnp.float32),
        mesh=mesh,
        scratch_types=[
            pltpu.VMEM((BPW,), jnp.int32),
            pltpu.VMEM((BPW,), jnp.int32),
            pltpu.VMEM((BPW,), jnp.int32),
            pltpu.VMEM((BPW,), jnp.float32),
            pltpu.VMEM((BPW,), jnp.float32),
            pltpu.VMEM((BPW,), jnp.int32),
            pltpu.VMEM((BPW,), jnp.float32),
            pltpu.VMEM((ROWPAD,), jnp.float32),
            pltpu.SemaphoreType.DMA,
        ],
        compiler_params=pltpu.CompilerParams(needs_layout_passes=False),
    )(flat, tgt)


def _logmean_body(r_ref, o_ref):
    o_ref[0, 0] = jnp.sum(jnp.log(r_ref[...])) * (1.0 / B)


@jax.jit
def _logmean(r):
    out = pl.pallas_call(
        _logmean_body,
        out_shape=jax.ShapeDtypeStruct((1, 1), jnp.float32),
        out_specs=pl.BlockSpec(memory_space=pltpu.SMEM),
    )(r.reshape(128, 128))
    return out[0, 0]


def kernel(inputs, targets):
    flat = inputs.reshape(B * C)
    tgt = targets.astype(jnp.int32)
    r = _sc_ratios(flat, tgt)
    return _logmean(r)


# speedup vs baseline: 2.2089x; 1.2207x over previous
import jax
import jax.numpy as jnp
from jax import lax
from jax.experimental import pallas as pl
from jax.experimental.pallas import tpu as pltpu
from jax.experimental.pallas import tpu_sc as plsc

B = 16384
C = 1000
NC = 2
NS = 16
NW = NC * NS
BPW = B // NW
NG = BPW // 16
NBUF = 4

_IN_BOUNDS = "promise_in_bounds"


def _rot(x, lanes, sh):
    return x.at[(lanes + sh) & 15].get(mode=_IN_BOUNDS)


def _allsum(x, lanes):
    for sh in (8, 4, 2, 1):
        x = x + _rot(x, lanes, sh)
    return x


def _sc_body(x_hbm, tgt_hbm, out_hbm,
             tgt_v, b0, b1, b2, b3, out_v, s0, s1, s2, s3):
    bufs = [b0, b1, b2, b3]
    sems = [s0, s1, s2, s3]
    wid = lax.axis_index("s") * NC + lax.axis_index("c")
    base = wid * BPW
    lanes = lax.iota(jnp.int32, 16)
    zeros16 = jnp.zeros((16,), jnp.int32)

    pltpu.sync_copy(tgt_hbm.at[pl.ds(base, BPW)], tgt_v)

    def start(g):
        p = g % NBUF
        return pltpu.async_copy(x_hbm.at[pl.ds(base + g * 16, 16)],
                                bufs[p], sems[p])

    handles = {}
    for g in range(NBUF - 1):
        handles[g] = start(g)

    for g in range(NG):
        if g + NBUF - 1 < NG:
            handles[g + NBUF - 1] = start(g + NBUF - 1)
        handles[g].wait()
        buf = bufs[g % NBUF]
        sl = pl.ds(g * 16, 16)
        t16 = tgt_v[sl]
        xt = plsc.load_gather(buf, [lanes, t16])
        x0 = plsc.load_gather(buf, [lanes, zeros16])
        out16 = 1.0 + jnp.exp(x0 - xt)
        out_v[sl] = out16
        zmask = t16 == 0
        nz = _allsum(zmask.astype(jnp.int32), lanes)[0]

        @pl.when(nz > 0)
        def _():
            def colbody(c, acc):
                col = plsc.load_gather(buf, [lanes,
                                             jnp.broadcast_to(c, (16,))])
                return acc + jnp.exp(col - x0)
            acc = lax.fori_loop(0, C, colbody, jnp.zeros((16,), jnp.float32))
            out_v[sl] = jnp.where(zmask, acc, out16)

    pltpu.sync_copy(out_v, out_hbm.at[pl.ds(base, BPW)])


@jax.jit
def _sc_ratios(x2d, tgt):
    mesh = plsc.VectorSubcoreMesh(core_axis_name="c", subcore_axis_name="s",
                                  num_cores=NC, num_subcores=NS)
    return pl.kernel(
        _sc_body,
        out_type=jax.ShapeDtypeStruct((B,), jnp.float32),
        mesh=mesh,
        scratch_types=[
            pltpu.VMEM((BPW,), jnp.int32),
            pltpu.VMEM((16, C), jnp.float32),
            pltpu.VMEM((16, C), jnp.float32),
            pltpu.VMEM((16, C), jnp.float32),
            pltpu.VMEM((16, C), jnp.float32),
            pltpu.VMEM((BPW,), jnp.float32),
            pltpu.SemaphoreType.DMA,
            pltpu.SemaphoreType.DMA,
            pltpu.SemaphoreType.DMA,
            pltpu.SemaphoreType.DMA,
        ],
        compiler_params=pltpu.CompilerParams(
            needs_layout_passes=False,
            use_tc_tiling_on_sc=True,
        ),
    )(x2d, tgt)


def _logmean_body(r_ref, o_ref):
    o_ref[0, 0] = jnp.sum(jnp.log(r_ref[...])) * (1.0 / B)


@jax.jit
def _logmean(r):
    out = pl.pallas_call(
        _logmean_body,
        out_shape=jax.ShapeDtypeStruct((1, 1), jnp.float32),
        out_specs=pl.BlockSpec(memory_space=pltpu.SMEM),
    )(r.reshape(128, 128))
    return out[0, 0]


def kernel(inputs, targets):
    tgt = targets.astype(jnp.int32)
    r = _sc_ratios(inputs, tgt)
    return _logmean(r)
